# Initial kernel scaffold; baseline (speedup 1.0000x reference)
import jax
import jax.numpy as jnp
from jax.experimental import pallas as pl


def kernel(features, edge_index, weight1, weight3):
    raise NotImplementedError("write your pallas kernel here")


# speedup vs baseline: 16.1773x; 16.1773x over previous
import functools

import jax
import jax.numpy as jnp
from jax import lax
from jax.experimental import pallas as pl
from jax.experimental.pallas import tpu as pltpu
from jax.experimental.pallas import tpu_sc as plsc

N_NODES = 10000
N_EDGES = 320000
D_FEAT = 128
HID = 16
N_CLASSES = 64

NC = 2
NS = 16
NW = NC * NS
CHUNK = 128
K_CHUNKS = 80
E_PAD = NW * K_CHUNKS * CHUNK
N_ACC = 10240
ZROWS = N_ACC // NS


def _spmm_sc(x, src_r, dst_r, zeros_hbm):
    mesh = plsc.VectorSubcoreMesh(core_axis_name="c", subcore_axis_name="s")

    @functools.partial(
        pl.kernel,
        mesh=mesh,
        out_type=jax.ShapeDtypeStruct((NC, N_ACC, HID), jnp.float32),
        compiler_params=pltpu.CompilerParams(use_tc_tiling_on_sc=False),
        scratch_types=[
            pltpu.VMEM((K_CHUNKS, CHUNK), jnp.int32),
            pltpu.VMEM((K_CHUNKS, CHUNK), jnp.int32),
            pltpu.VMEM((CHUNK, HID), jnp.float32),
            pltpu.VMEM((CHUNK, HID), jnp.float32),
            pltpu.VMEM_SHARED((N_ACC, HID), jnp.float32),
            pltpu.SemaphoreType.DMA,
            pltpu.SemaphoreType.DMA,
        ],
    )
    def spmm(x_hbm, src_hbm, dst_hbm, z_hbm, out_hbm,
             src_v, dst_v, rows0, rows1, acc, gsem0, gsem1):
        c = lax.axis_index("c")
        s = lax.axis_index("s")
        wid = c * NS + s
        rows = (rows0, rows1)
        gsems = (gsem0, gsem1)

        pltpu.sync_copy(z_hbm, acc.at[pl.ds(s * ZROWS, ZROWS)])
        pltpu.sync_copy(src_hbm.at[wid], src_v)
        pltpu.sync_copy(dst_hbm.at[wid], dst_v)
        plsc.subcore_barrier()

        for b in range(2):
            pltpu.async_copy(x_hbm.at[src_v.at[b]], rows[b], gsems[b])

        def step(g, carry):
            for b in range(2):
                j = g * 2 + b
                pltpu.make_async_copy(
                    x_hbm.at[src_v.at[j]], rows[b], gsems[b]).wait()
                pltpu.sync_copy(rows[b], acc.at[dst_v.at[j]], add=True)

                @pl.when(j + 2 < K_CHUNKS)
                def _():
                    pltpu.async_copy(
                        x_hbm.at[src_v.at[j + 2]], rows[b], gsems[b])
            return carry

        lax.fori_loop(0, K_CHUNKS // 2, step, 0)

        plsc.subcore_barrier()
        pltpu.sync_copy(acc.at[pl.ds(s * ZROWS, ZROWS)],
                        out_hbm.at[c, pl.ds(s * ZROWS, ZROWS)])

    return spmm(x, src_r, dst_r, zeros_hbm)


def _project(features, w1):
    def body(f_ref, w_ref, o_ref):
        o_ref[...] = jnp.dot(f_ref[...], w_ref[...],
                             preferred_element_type=jnp.float32)

    return pl.pallas_call(
        body,
        grid=(5,),
        in_specs=[
            pl.BlockSpec((2000, D_FEAT), lambda i: (i, 0)),
            pl.BlockSpec((D_FEAT, HID), lambda i: (0, 0)),
        ],
        out_specs=pl.BlockSpec((2000, HID), lambda i: (i, 0)),
        out_shape=jax.ShapeDtypeStruct((N_NODES, HID), jnp.float32),
    )(features, w1)


def _combine_relu(parts):
    def body(p_ref, o_ref):
        o_ref[...] = jnp.maximum(p_ref[0] + p_ref[1], 0.0)

    return pl.pallas_call(
        body,
        grid=(5,),
        in_specs=[pl.BlockSpec((NC, 2000, HID), lambda i: (0, i, 0))],
        out_specs=pl.BlockSpec((2000, HID), lambda i: (i, 0)),
        out_shape=jax.ShapeDtypeStruct((N_NODES, HID), jnp.float32),
    )(parts)


def _combine_matmul(parts, w3):
    def body(p_ref, w_ref, o_ref):
        z = p_ref[0] + p_ref[1]
        o_ref[...] = jnp.dot(z, w_ref[...],
                             preferred_element_type=jnp.float32)

    return pl.pallas_call(
        body,
        grid=(5,),
        in_specs=[
            pl.BlockSpec((NC, 2000, HID), lambda i: (0, i, 0)),
            pl.BlockSpec((HID, N_CLASSES), lambda i: (0, 0)),
        ],
        out_specs=pl.BlockSpec((2000, N_CLASSES), lambda i: (i, 0)),
        out_shape=jax.ShapeDtypeStruct((N_NODES, N_CLASSES), jnp.float32),
    )(parts, w3)


def kernel(features, edge_index, weight1, weight3):
    pad = E_PAD - N_EDGES
    src = jnp.concatenate([edge_index[0], jnp.zeros((pad,), jnp.int32)])
    dst = jnp.concatenate([edge_index[1],
                           jnp.full((pad,), N_NODES, jnp.int32)])
    src_r = src.reshape(NW, K_CHUNKS, CHUNK)
    dst_r = dst.reshape(NW, K_CHUNKS, CHUNK)
    zeros_hbm = jnp.zeros((ZROWS, HID), jnp.float32)

    x1 = _project(features, weight1)
    p1 = _spmm_sc(x1, src_r, dst_r, zeros_hbm)
    h = _combine_relu(p1)
    p2 = _spmm_sc(h, src_r, dst_r, zeros_hbm)
    return _combine_matmul(p2, weight3)


# speedup vs baseline: 17.0426x; 1.0535x over previous
import functools

import jax
import jax.numpy as jnp
from jax import lax
from jax.experimental import pallas as pl
from jax.experimental.pallas import tpu as pltpu
from jax.experimental.pallas import tpu_sc as plsc

N_NODES = 10000
N_EDGES = 320000
D_FEAT = 128
HID = 16
N_CLASSES = 64

NC = 2
NS = 16
NW = NC * NS
CHUNK = 128
K_CHUNKS = 80
E_PAD = NW * K_CHUNKS * CHUNK
N_ACC = 10240
ZROWS = N_ACC // NS
NBUF = 8
LAG = 4


def _spmm_body(x_src, src_v, dst_v, rows, gsems, ssems, acc):
    def gather_wait(j, b):
        pltpu.make_async_copy(
            x_src.at[src_v.at[j]], rows.at[b], gsems.at[b]).wait()

    def scatter_wait(j, b):
        pltpu.make_async_copy(
            rows.at[b], acc.at[dst_v.at[j]], ssems.at[b]).wait()

    for b in range(LAG):
        pltpu.async_copy(x_src.at[src_v.at[b]], rows.at[b], gsems.at[b])

    def step(g, carry):
        for b in range(NBUF):
            j = g * NBUF + b
            gather_wait(j, b)
            pltpu.async_copy(rows.at[b], acc.at[dst_v.at[j]], ssems.at[b],
                             add=True)
            bn = (b + LAG) % NBUF

            @pl.when(j + LAG < K_CHUNKS)
            def _():
                @pl.when(j >= LAG)
                def _():
                    scatter_wait(j - LAG, bn)

                pltpu.async_copy(
                    x_src.at[src_v.at[j + LAG]], rows.at[bn], gsems.at[bn])

        return carry

    lax.fori_loop(0, K_CHUNKS // NBUF, step, 0)
    for j in range(K_CHUNKS - 2 * LAG, K_CHUNKS):
        scatter_wait(j, j % NBUF)


def _sc_scratch():
    return [
        pltpu.VMEM((K_CHUNKS, CHUNK), jnp.int32),
        pltpu.VMEM((K_CHUNKS, CHUNK), jnp.int32),
        pltpu.VMEM((NBUF, CHUNK, HID), jnp.float32),
        pltpu.VMEM_SHARED((N_ACC, HID), jnp.float32),
        pltpu.SemaphoreType.DMA((NBUF,)),
        pltpu.SemaphoreType.DMA((NBUF,)),
    ]


def _spmm_sc1(x, src_r, dst_r, zeros_hbm):
    mesh = plsc.VectorSubcoreMesh(core_axis_name="c", subcore_axis_name="s")

    @functools.partial(
        pl.kernel,
        mesh=mesh,
        out_type=jax.ShapeDtypeStruct((NC, N_ACC, HID), jnp.float32),
        scratch_types=_sc_scratch(),
        compiler_params=pltpu.CompilerParams(use_tc_tiling_on_sc=False),
    )
    def spmm(x_hbm, src_hbm, dst_hbm, z_hbm, out_hbm,
             src_v, dst_v, rows, acc, gsems, ssems):
        c = lax.axis_index("c")
        s = lax.axis_index("s")
        wid = c * NS + s

        pltpu.sync_copy(z_hbm, acc.at[pl.ds(s * ZROWS, ZROWS)])
        pltpu.sync_copy(src_hbm.at[wid], src_v)
        pltpu.sync_copy(dst_hbm.at[wid], dst_v)
        plsc.subcore_barrier()

        _spmm_body(x_hbm, src_v, dst_v, rows, gsems, ssems, acc)

        plsc.subcore_barrier()
        pltpu.sync_copy(acc.at[pl.ds(s * ZROWS, ZROWS)],
                        out_hbm.at[c, pl.ds(s * ZROWS, ZROWS)])

    return spmm(x, src_r, dst_r, zeros_hbm)


def _spmm_sc2(parts, src_r, dst_r, zeros_hbm):
    mesh = plsc.VectorSubcoreMesh(core_axis_name="c", subcore_axis_name="s")

    @functools.partial(
        pl.kernel,
        mesh=mesh,
        out_type=jax.ShapeDtypeStruct((NC, N_ACC, HID), jnp.float32),
        scratch_types=_sc_scratch() + [
            pltpu.VMEM_SHARED((N_ACC, HID), jnp.float32),
            pltpu.VMEM((ZROWS, HID), jnp.float32),
            pltpu.VMEM((ZROWS, HID), jnp.float32),
        ],
        compiler_params=pltpu.CompilerParams(use_tc_tiling_on_sc=False),
    )
    def spmm(p_hbm, src_hbm, dst_hbm, z_hbm, out_hbm,
             src_v, dst_v, rows, acc, gsems, ssems, htab, p0_v, p1_v):
        c = lax.axis_index("c")
        s = lax.axis_index("s")
        wid = c * NS + s

        pltpu.sync_copy(z_hbm, acc.at[pl.ds(s * ZROWS, ZROWS)])
        pltpu.sync_copy(src_hbm.at[wid], src_v)
        pltpu.sync_copy(dst_hbm.at[wid], dst_v)

        pltpu.sync_copy(p_hbm.at[0, pl.ds(s * ZROWS, ZROWS)], p0_v)
        pltpu.sync_copy(p_hbm.at[1, pl.ds(s * ZROWS, ZROWS)], p1_v)

        def relu_row(i, carry):
            p0_v[i, :] = jnp.maximum(p0_v[i, :] + p1_v[i, :], 0.0)
            return carry

        lax.fori_loop(0, ZROWS, relu_row, 0)
        pltpu.sync_copy(p0_v, htab.at[pl.ds(s * ZROWS, ZROWS)])
        plsc.subcore_barrier()

        _spmm_body(htab, src_v, dst_v, rows, gsems, ssems, acc)

        plsc.subcore_barrier()
        pltpu.sync_copy(acc.at[pl.ds(s * ZROWS, ZROWS)],
                        out_hbm.at[c, pl.ds(s * ZROWS, ZROWS)])

    return spmm(parts, src_r, dst_r, zeros_hbm)


def _project(features, w1):
    def body(f_ref, w_ref, o_ref):
        o_ref[...] = jnp.dot(f_ref[...], w_ref[...],
                             preferred_element_type=jnp.float32)

    return pl.pallas_call(
        body,
        grid=(5,),
        in_specs=[
            pl.BlockSpec((2000, D_FEAT), lambda i: (i, 0)),
            pl.BlockSpec((D_FEAT, HID), lambda i: (0, 0)),
        ],
        out_specs=pl.BlockSpec((2000, HID), lambda i: (i, 0)),
        out_shape=jax.ShapeDtypeStruct((N_NODES, HID), jnp.float32),
    )(features, w1)


def _relu_combine(parts):
    def body(p_ref, o_ref):
        o_ref[...] = jnp.maximum(p_ref[0] + p_ref[1], 0.0)

    return pl.pallas_call(
        body,
        grid=(5,),
        in_specs=[pl.BlockSpec((NC, N_ACC // 5, HID), lambda i: (0, i, 0))],
        out_specs=pl.BlockSpec((N_ACC // 5, HID), lambda i: (i, 0)),
        out_shape=jax.ShapeDtypeStruct((N_ACC, HID), jnp.float32),
    )(parts)


def _combine_matmul(parts, w3):
    def body(p_ref, w_ref, o_ref):
        z = p_ref[0] + p_ref[1]
        o_ref[...] = jnp.dot(z, w_ref[...],
                             preferred_element_type=jnp.float32)

    return pl.pallas_call(
        body,
        grid=(5,),
        in_specs=[
            pl.BlockSpec((NC, 2000, HID), lambda i: (0, i, 0)),
            pl.BlockSpec((HID, N_CLASSES), lambda i: (0, 0)),
        ],
        out_specs=pl.BlockSpec((2000, N_CLASSES), lambda i: (i, 0)),
        out_shape=jax.ShapeDtypeStruct((N_NODES, N_CLASSES), jnp.float32),
    )(parts, w3)


def kernel(features, edge_index, weight1, weight3):
    pad = E_PAD - N_EDGES
    junk = N_NODES + jnp.arange(pad, dtype=jnp.int32) % (N_ACC - N_NODES)
    src = jnp.concatenate([edge_index[0], jnp.zeros((pad,), jnp.int32)])
    dst = jnp.concatenate([edge_index[1], junk])
    src_r = src.reshape(NW, K_CHUNKS, CHUNK)
    dst_r = dst.reshape(NW, K_CHUNKS, CHUNK)
    zeros_hbm = jnp.zeros((ZROWS, HID), jnp.float32)

    x1 = _project(features, weight1)
    p1 = _spmm_sc1(x1, src_r, dst_r, zeros_hbm)
    h = _relu_combine(p1)
    p2 = _spmm_sc1(h, src_r, dst_r, zeros_hbm)
    return _combine_matmul(p2, weight3)


# speedup vs baseline: 22.4032x; 1.3145x over previous
import functools

import jax
import jax.numpy as jnp
from jax import lax
from jax.experimental import pallas as pl
from jax.experimental.pallas import tpu as pltpu
from jax.experimental.pallas import tpu_sc as plsc

N_NODES = 10000
N_EDGES = 320000
D_FEAT = 128
HID = 16
N_CLASSES = 64

NC = 2
NS = 16
NW = NC * NS
CHUNK = 128
K_CHUNKS = 80
E_PAD = NW * K_CHUNKS * CHUNK
N_ACC = 10240
ZROWS = N_ACC // NS
NBUF = 8
LAG = 4


def _spmm_body(x_src, src_v, dst_v, rows, gsems, ssems, acc):
    def gather_wait(j, b):
        pltpu.make_async_copy(
            x_src.at[src_v.at[j]], rows.at[b], gsems.at[b]).wait()

    def scatter_wait(j, b):
        pltpu.make_async_copy(
            rows.at[b], acc.at[dst_v.at[j]], ssems.at[b]).wait()

    for b in range(LAG):
        pltpu.async_copy(x_src.at[src_v.at[b]], rows.at[b], gsems.at[b])

    def step(g, carry):
        for b in range(NBUF):
            j = g * NBUF + b
            gather_wait(j, b)
            pltpu.async_copy(rows.at[b], acc.at[dst_v.at[j]], ssems.at[b],
                             add=True)
            bn = (b + LAG) % NBUF

            @pl.when(j + LAG < K_CHUNKS)
            def _():
                @pl.when(j >= LAG)
                def _():
                    scatter_wait(j - LAG, bn)

                pltpu.async_copy(
                    x_src.at[src_v.at[j + LAG]], rows.at[bn], gsems.at[bn])

        return carry

    lax.fori_loop(0, K_CHUNKS // NBUF, step, 0)
    for j in range(K_CHUNKS - 2 * LAG, K_CHUNKS):
        scatter_wait(j, j % NBUF)


def _sc_scratch():
    return [
        pltpu.VMEM((K_CHUNKS, CHUNK), jnp.int32),
        pltpu.VMEM((K_CHUNKS, CHUNK), jnp.int32),
        pltpu.VMEM((NBUF, CHUNK, HID), jnp.float32),
        pltpu.VMEM_SHARED((N_ACC, HID), jnp.float32),
        pltpu.SemaphoreType.DMA((NBUF,)),
        pltpu.SemaphoreType.DMA((NBUF,)),
    ]


def _spmm_sc1(x, src_r, dst_r, zeros_hbm):
    mesh = plsc.VectorSubcoreMesh(core_axis_name="c", subcore_axis_name="s")

    @functools.partial(
        pl.kernel,
        mesh=mesh,
        out_type=jax.ShapeDtypeStruct((NC, N_ACC, HID), jnp.float32),
        scratch_types=_sc_scratch(),
        compiler_params=pltpu.CompilerParams(use_tc_tiling_on_sc=False),
    )
    def spmm(x_hbm, src_hbm, dst_hbm, z_hbm, out_hbm,
             src_v, dst_v, rows, acc, gsems, ssems):
        c = lax.axis_index("c")
        s = lax.axis_index("s")
        wid = c * NS + s

        pltpu.sync_copy(z_hbm, acc.at[pl.ds(s * ZROWS, ZROWS)])
        pltpu.sync_copy(src_hbm.at[wid], src_v)
        pltpu.sync_copy(dst_hbm.at[wid], dst_v)
        plsc.subcore_barrier()

        _spmm_body(x_hbm, src_v, dst_v, rows, gsems, ssems, acc)

        plsc.subcore_barrier()
        pltpu.sync_copy(acc.at[pl.ds(s * ZROWS, ZROWS)],
                        out_hbm.at[c, pl.ds(s * ZROWS, ZROWS)])

    return spmm(x, src_r, dst_r, zeros_hbm)


def _spmm_sc2(parts, src_r, dst_r, zeros_hbm):
    mesh = plsc.VectorSubcoreMesh(core_axis_name="c", subcore_axis_name="s")

    @functools.partial(
        pl.kernel,
        mesh=mesh,
        out_type=jax.ShapeDtypeStruct((NC, N_ACC, HID), jnp.float32),
        scratch_types=_sc_scratch() + [
            pltpu.VMEM_SHARED((N_ACC, HID), jnp.float32),
            pltpu.VMEM((ZROWS, HID), jnp.float32),
            pltpu.VMEM((ZROWS, HID), jnp.float32),
        ],
        compiler_params=pltpu.CompilerParams(use_tc_tiling_on_sc=False),
    )
    def spmm(p_hbm, src_hbm, dst_hbm, z_hbm, out_hbm,
             src_v, dst_v, rows, acc, gsems, ssems, htab, p0_v, p1_v):
        c = lax.axis_index("c")
        s = lax.axis_index("s")
        wid = c * NS + s

        pltpu.sync_copy(z_hbm, acc.at[pl.ds(s * ZROWS, ZROWS)])
        pltpu.sync_copy(src_hbm.at[wid], src_v)
        pltpu.sync_copy(dst_hbm.at[wid], dst_v)

        pltpu.sync_copy(p_hbm.at[0, pl.ds(s * ZROWS, ZROWS)], p0_v)
        pltpu.sync_copy(p_hbm.at[1, pl.ds(s * ZROWS, ZROWS)], p1_v)

        def relu_row(i, carry):
            p0_v[i, :] = jnp.maximum(p0_v[i, :] + p1_v[i, :], 0.0)
            return carry

        lax.fori_loop(0, ZROWS, relu_row, 0)
        pltpu.sync_copy(p0_v, htab.at[pl.ds(s * ZROWS, ZROWS)])
        plsc.subcore_barrier()

        _spmm_body(htab, src_v, dst_v, rows, gsems, ssems, acc)

        plsc.subcore_barrier()
        pltpu.sync_copy(acc.at[pl.ds(s * ZROWS, ZROWS)],
                        out_hbm.at[c, pl.ds(s * ZROWS, ZROWS)])

    return spmm(parts, src_r, dst_r, zeros_hbm)


def _project(features, w1):
    def body(f_ref, w_ref, o_ref):
        o_ref[...] = jnp.dot(f_ref[...], w_ref[...],
                             preferred_element_type=jnp.float32)

    return pl.pallas_call(
        body,
        grid=(5,),
        in_specs=[
            pl.BlockSpec((2000, D_FEAT), lambda i: (i, 0)),
            pl.BlockSpec((D_FEAT, HID), lambda i: (0, 0)),
        ],
        out_specs=pl.BlockSpec((2000, HID), lambda i: (i, 0)),
        out_shape=jax.ShapeDtypeStruct((N_NODES, HID), jnp.float32),
    )(features, w1)


def _relu_combine(parts):
    def body(p_ref, o_ref):
        o_ref[...] = jnp.maximum(p_ref[0] + p_ref[1], 0.0)

    return pl.pallas_call(
        body,
        grid=(5,),
        in_specs=[pl.BlockSpec((NC, N_ACC // 5, HID), lambda i: (0, i, 0))],
        out_specs=pl.BlockSpec((N_ACC // 5, HID), lambda i: (i, 0)),
        out_shape=jax.ShapeDtypeStruct((N_ACC, HID), jnp.float32),
    )(parts)


def _combine_matmul(parts, w3):
    def body(p_ref, w_ref, o_ref):
        z = p_ref[0] + p_ref[1]
        o_ref[...] = jnp.dot(z, w_ref[...],
                             preferred_element_type=jnp.float32)

    return pl.pallas_call(
        body,
        grid=(5,),
        in_specs=[
            pl.BlockSpec((NC, 2000, HID), lambda i: (0, i, 0)),
            pl.BlockSpec((HID, N_CLASSES), lambda i: (0, 0)),
        ],
        out_specs=pl.BlockSpec((2000, N_CLASSES), lambda i: (i, 0)),
        out_shape=jax.ShapeDtypeStruct((N_NODES, N_CLASSES), jnp.float32),
    )(parts, w3)


def kernel(features, edge_index, weight1, weight3):
    pad = E_PAD - N_EDGES
    junk = N_NODES + jnp.arange(pad, dtype=jnp.int32) % (N_ACC - N_NODES)
    src = jnp.concatenate([edge_index[0], jnp.zeros((pad,), jnp.int32)])
    dst = jnp.concatenate([edge_index[1], junk])
    src_r = src.reshape(NW, K_CHUNKS, CHUNK)
    dst_r = dst.reshape(NW, K_CHUNKS, CHUNK)
    zeros_hbm = jnp.zeros((ZROWS, HID), jnp.float32)

    x1 = _project(features, weight1)
    p1 = _spmm_sc1(x1, src_r, dst_r, zeros_hbm)
    p2 = _spmm_sc2(p1, src_r, dst_r, zeros_hbm)
    return _combine_matmul(p2, weight3)


# speedup vs baseline: 26.8608x; 1.1990x over previous
import functools

import jax
import jax.numpy as jnp
from jax import lax
from jax.experimental import pallas as pl
from jax.experimental.pallas import tpu as pltpu
from jax.experimental.pallas import tpu_sc as plsc

N_NODES = 10000
N_EDGES = 320000
D_FEAT = 128
HID = 16
N_CLASSES = 64

NC = 2
NS = 16
NW = NC * NS
CHUNK = 128
ROWS_E = N_EDGES // CHUNK
K_BASE = ROWS_E // NW
N_XTRA = ROWS_E - K_BASE * NW
N_ACC = 10240
ZROWS = N_ACC // NS
NBUF = 6
LAG = 3


def _spmm_body(x_src, src_v, dst_v, rows, gsems, ssems, acc, has_extra):
    def gather_wait(j, b):
        pltpu.make_async_copy(
            x_src.at[src_v.at[j]], rows.at[b], gsems.at[b]).wait()

    def scatter_wait(j, b):
        pltpu.make_async_copy(
            rows.at[b], acc.at[dst_v.at[j]], ssems.at[b]).wait()

    for b in range(LAG):
        pltpu.async_copy(x_src.at[src_v.at[b]], rows.at[b], gsems.at[b])

    def step(g, carry):
        for b in range(NBUF):
            j = g * NBUF + b
            gather_wait(j, b)
            pltpu.async_copy(rows.at[b], acc.at[dst_v.at[j]], ssems.at[b],
                             add=True)
            bn = (b + LAG) % NBUF

            @pl.when(j + LAG < K_BASE)
            def _():
                @pl.when(j >= LAG)
                def _():
                    scatter_wait(j - LAG, bn)

                pltpu.async_copy(
                    x_src.at[src_v.at[j + LAG]], rows.at[bn], gsems.at[bn])

        return carry

    lax.fori_loop(0, K_BASE // NBUF, step, 0)
    for j in range(K_BASE - 2 * LAG, K_BASE):
        scatter_wait(j, j % NBUF)

    @pl.when(has_extra)
    def _():
        pltpu.async_copy(x_src.at[src_v.at[K_BASE]], rows.at[0], gsems.at[0])
        gather_wait(K_BASE, 0)
        pltpu.async_copy(rows.at[0], acc.at[dst_v.at[K_BASE]], ssems.at[0],
                         add=True)
        scatter_wait(K_BASE, 0)


def _sc_scratch():
    return [
        pltpu.VMEM((K_BASE + 1, CHUNK), jnp.int32),
        pltpu.VMEM((K_BASE + 1, CHUNK), jnp.int32),
        pltpu.VMEM((NBUF, CHUNK, HID), jnp.float32),
        pltpu.VMEM_SHARED((N_ACC, HID), jnp.float32),
        pltpu.SemaphoreType.DMA((NBUF,)),
        pltpu.SemaphoreType.DMA((NBUF,)),
    ]


def _stage_indices(src_hbm, dst_hbm, src_v, dst_v, wid):
    row0 = wid * K_BASE + jnp.minimum(wid, N_XTRA)
    pltpu.sync_copy(src_hbm.at[pl.ds(row0, K_BASE)],
                    src_v.at[pl.ds(0, K_BASE)])
    pltpu.sync_copy(dst_hbm.at[pl.ds(row0, K_BASE)],
                    dst_v.at[pl.ds(0, K_BASE)])

    @pl.when(wid < N_XTRA)
    def _():
        pltpu.sync_copy(src_hbm.at[pl.ds(row0 + K_BASE, 1)],
                        src_v.at[pl.ds(K_BASE, 1)])
        pltpu.sync_copy(dst_hbm.at[pl.ds(row0 + K_BASE, 1)],
                        dst_v.at[pl.ds(K_BASE, 1)])


def _spmm_sc1(x, src_r, dst_r, zeros_hbm):
    mesh = plsc.VectorSubcoreMesh(core_axis_name="c", subcore_axis_name="s")

    @functools.partial(
        pl.kernel,
        mesh=mesh,
        out_type=jax.ShapeDtypeStruct((NC, N_ACC, HID), jnp.float32),
        scratch_types=_sc_scratch(),
        compiler_params=pltpu.CompilerParams(use_tc_tiling_on_sc=False),
    )
    def spmm(x_hbm, src_hbm, dst_hbm, z_hbm, out_hbm,
             src_v, dst_v, rows, acc, gsems, ssems):
        c = lax.axis_index("c")
        s = lax.axis_index("s")
        wid = c * NS + s

        pltpu.sync_copy(z_hbm, acc.at[pl.ds(s * ZROWS, ZROWS)])
        _stage_indices(src_hbm, dst_hbm, src_v, dst_v, wid)
        plsc.subcore_barrier()

        _spmm_body(x_hbm, src_v, dst_v, rows, gsems, ssems, acc,
                   wid < N_XTRA)

        plsc.subcore_barrier()
        pltpu.sync_copy(acc.at[pl.ds(s * ZROWS, ZROWS)],
                        out_hbm.at[c, pl.ds(s * ZROWS, ZROWS)])

    return spmm(x, src_r, dst_r, zeros_hbm)


def _spmm_sc2(parts, src_r, dst_r, zeros_hbm):
    mesh = plsc.VectorSubcoreMesh(core_axis_name="c", subcore_axis_name="s")

    @functools.partial(
        pl.kernel,
        mesh=mesh,
        out_type=jax.ShapeDtypeStruct((NC, N_ACC, HID), jnp.float32),
        scratch_types=_sc_scratch() + [
            pltpu.VMEM_SHARED((N_ACC, HID), jnp.float32),
            pltpu.VMEM((ZROWS, HID), jnp.float32),
            pltpu.VMEM((ZROWS, HID), jnp.float32),
        ],
        compiler_params=pltpu.CompilerParams(use_tc_tiling_on_sc=False),
    )
    def spmm(p_hbm, src_hbm, dst_hbm, z_hbm, out_hbm,
             src_v, dst_v, rows, acc, gsems, ssems, htab, p0_v, p1_v):
        c = lax.axis_index("c")
        s = lax.axis_index("s")
        wid = c * NS + s

        pltpu.sync_copy(z_hbm, acc.at[pl.ds(s * ZROWS, ZROWS)])
        _stage_indices(src_hbm, dst_hbm, src_v, dst_v, wid)

        pltpu.sync_copy(p_hbm.at[0, pl.ds(s * ZROWS, ZROWS)], p0_v)
        pltpu.sync_copy(p_hbm.at[1, pl.ds(s * ZROWS, ZROWS)], p1_v)

        def relu_row(i, carry):
            p0_v[i, :] = jnp.maximum(p0_v[i, :] + p1_v[i, :], 0.0)
            return carry

        lax.fori_loop(0, ZROWS, relu_row, 0)
        pltpu.sync_copy(p0_v, htab.at[pl.ds(s * ZROWS, ZROWS)])
        plsc.subcore_barrier()

        _spmm_body(htab, src_v, dst_v, rows, gsems, ssems, acc,
                   wid < N_XTRA)

        plsc.subcore_barrier()
        pltpu.sync_copy(acc.at[pl.ds(s * ZROWS, ZROWS)],
                        out_hbm.at[c, pl.ds(s * ZROWS, ZROWS)])

    return spmm(parts, src_r, dst_r, zeros_hbm)


def _project(features, w1):
    def body(f_ref, w_ref, o_ref):
        o_ref[...] = jnp.dot(f_ref[...], w_ref[...],
                             preferred_element_type=jnp.float32)

    return pl.pallas_call(
        body,
        grid=(5,),
        in_specs=[
            pl.BlockSpec((2000, D_FEAT), lambda i: (i, 0)),
            pl.BlockSpec((D_FEAT, HID), lambda i: (0, 0)),
        ],
        out_specs=pl.BlockSpec((2000, HID), lambda i: (i, 0)),
        out_shape=jax.ShapeDtypeStruct((N_NODES, HID), jnp.float32),
    )(features, w1)


def _combine_matmul(parts, w3):
    def body(p_ref, w_ref, o_ref):
        z = p_ref[0] + p_ref[1]
        o_ref[...] = jnp.dot(z, w_ref[...],
                             preferred_element_type=jnp.float32)

    return pl.pallas_call(
        body,
        grid=(5,),
        in_specs=[
            pl.BlockSpec((NC, 2000, HID), lambda i: (0, i, 0)),
            pl.BlockSpec((HID, N_CLASSES), lambda i: (0, 0)),
        ],
        out_specs=pl.BlockSpec((2000, N_CLASSES), lambda i: (i, 0)),
        out_shape=jax.ShapeDtypeStruct((N_NODES, N_CLASSES), jnp.float32),
    )(parts, w3)


def kernel(features, edge_index, weight1, weight3):
    src_r = edge_index[0].reshape(ROWS_E, CHUNK)
    dst_r = edge_index[1].reshape(ROWS_E, CHUNK)
    zeros_hbm = jnp.zeros((ZROWS, HID), jnp.float32)

    x1 = _project(features, weight1)
    p1 = _spmm_sc1(x1, src_r, dst_r, zeros_hbm)
    p2 = _spmm_sc2(p1, src_r, dst_r, zeros_hbm)
    return _combine_matmul(p2, weight3)


# speedup vs baseline: 29.2773x; 1.0900x over previous
import functools

import jax
import jax.numpy as jnp
from jax import lax
from jax.experimental import pallas as pl
from jax.experimental.pallas import tpu as pltpu
from jax.experimental.pallas import tpu_sc as plsc

N_NODES = 10000
N_EDGES = 320000
D_FEAT = 128
HID = 16
N_CLASSES = 64

NC = 2
NS = 16
NW = NC * NS
CHUNK = 128
ROWS_E = N_EDGES // CHUNK
K_BASE = ROWS_E // NW
N_XTRA = ROWS_E - K_BASE * NW
N_ACC = 10240
ZROWS = N_ACC // NS
NBUF = 6
LAG = 3


def _spmm_body(x_src, src_v, dst_v, rows, gsems, ssems, acc, has_extra):
    def gather_wait(j, b):
        pltpu.make_async_copy(
            x_src.at[src_v.at[j]], rows.at[b], gsems.at[b]).wait()

    def scatter_wait(j, b):
        pltpu.make_async_copy(
            rows.at[b], acc.at[dst_v.at[j]], ssems.at[b]).wait()

    for b in range(LAG):
        pltpu.async_copy(x_src.at[src_v.at[b]], rows.at[b], gsems.at[b])

    def step(g, carry):
        for b in range(NBUF):
            j = g * NBUF + b
            gather_wait(j, b)
            pltpu.async_copy(rows.at[b], acc.at[dst_v.at[j]], ssems.at[b],
                             add=True)
            bn = (b + LAG) % NBUF

            @pl.when(j + LAG < K_BASE)
            def _():
                @pl.when(j >= LAG)
                def _():
                    scatter_wait(j - LAG, bn)

                pltpu.async_copy(
                    x_src.at[src_v.at[j + LAG]], rows.at[bn], gsems.at[bn])

        return carry

    lax.fori_loop(0, K_BASE // NBUF, step, 0)
    for j in range(K_BASE - 2 * LAG, K_BASE):
        scatter_wait(j, j % NBUF)

    @pl.when(has_extra)
    def _():
        pltpu.async_copy(x_src.at[src_v.at[K_BASE]], rows.at[0], gsems.at[0])
        gather_wait(K_BASE, 0)
        pltpu.async_copy(rows.at[0], acc.at[dst_v.at[K_BASE]], ssems.at[0],
                         add=True)
        scatter_wait(K_BASE, 0)


def _sc_scratch():
    return [
        pltpu.VMEM((K_BASE + 1, CHUNK), jnp.int32),
        pltpu.VMEM((K_BASE + 1, CHUNK), jnp.int32),
        pltpu.VMEM((NBUF, CHUNK, HID), jnp.float32),
        pltpu.VMEM_SHARED((N_ACC, HID), jnp.float32),
        pltpu.SemaphoreType.DMA((NBUF,)),
        pltpu.SemaphoreType.DMA((NBUF,)),
    ]


def _stage_indices(e_hbm, src_v, dst_v, wid):
    row0 = wid * K_BASE + jnp.minimum(wid, N_XTRA)
    pltpu.sync_copy(e_hbm.at[0, pl.ds(row0, K_BASE)],
                    src_v.at[pl.ds(0, K_BASE)])
    pltpu.sync_copy(e_hbm.at[1, pl.ds(row0, K_BASE)],
                    dst_v.at[pl.ds(0, K_BASE)])

    @pl.when(wid < N_XTRA)
    def _():
        pltpu.sync_copy(e_hbm.at[0, pl.ds(row0 + K_BASE, 1)],
                        src_v.at[pl.ds(K_BASE, 1)])
        pltpu.sync_copy(e_hbm.at[1, pl.ds(row0 + K_BASE, 1)],
                        dst_v.at[pl.ds(K_BASE, 1)])


def _spmm_sc1(x, edges, zeros_hbm):
    mesh = plsc.VectorSubcoreMesh(core_axis_name="c", subcore_axis_name="s")

    @functools.partial(
        pl.kernel,
        mesh=mesh,
        out_type=jax.ShapeDtypeStruct((NC, N_ACC, HID), jnp.float32),
        scratch_types=_sc_scratch(),
        compiler_params=pltpu.CompilerParams(use_tc_tiling_on_sc=False),
    )
    def spmm(x_hbm, e_hbm, z_hbm, out_hbm,
             src_v, dst_v, rows, acc, gsems, ssems):
        c = lax.axis_index("c")
        s = lax.axis_index("s")
        wid = c * NS + s

        pltpu.sync_copy(z_hbm, acc.at[pl.ds(s * ZROWS, ZROWS)])
        _stage_indices(e_hbm, src_v, dst_v, wid)
        plsc.subcore_barrier()

        _spmm_body(x_hbm, src_v, dst_v, rows, gsems, ssems, acc,
                   wid < N_XTRA)

        plsc.subcore_barrier()
        pltpu.sync_copy(acc.at[pl.ds(s * ZROWS, ZROWS)],
                        out_hbm.at[c, pl.ds(s * ZROWS, ZROWS)])

    return spmm(x, edges, zeros_hbm)


def _spmm_sc2(parts, edges, zeros_hbm):
    mesh = plsc.VectorSubcoreMesh(core_axis_name="c", subcore_axis_name="s")

    @functools.partial(
        pl.kernel,
        mesh=mesh,
        out_type=jax.ShapeDtypeStruct((NC, N_ACC, HID), jnp.float32),
        scratch_types=_sc_scratch() + [
            pltpu.VMEM_SHARED((N_ACC, HID), jnp.float32),
            pltpu.VMEM((ZROWS, HID), jnp.float32),
            pltpu.VMEM((ZROWS, HID), jnp.float32),
        ],
        compiler_params=pltpu.CompilerParams(use_tc_tiling_on_sc=False),
    )
    def spmm(p_hbm, e_hbm, z_hbm, out_hbm,
             src_v, dst_v, rows, acc, gsems, ssems, htab, p0_v, p1_v):
        c = lax.axis_index("c")
        s = lax.axis_index("s")
        wid = c * NS + s

        pltpu.sync_copy(z_hbm, acc.at[pl.ds(s * ZROWS, ZROWS)])
        _stage_indices(e_hbm, src_v, dst_v, wid)

        pltpu.sync_copy(p_hbm.at[0, pl.ds(s * ZROWS, ZROWS)], p0_v)
        pltpu.sync_copy(p_hbm.at[1, pl.ds(s * ZROWS, ZROWS)], p1_v)

        def relu_row(i, carry):
            p0_v[i, :] = jnp.maximum(p0_v[i, :] + p1_v[i, :], 0.0)
            return carry

        lax.fori_loop(0, ZROWS, relu_row, 0)
        pltpu.sync_copy(p0_v, htab.at[pl.ds(s * ZROWS, ZROWS)])
        plsc.subcore_barrier()

        _spmm_body(htab, src_v, dst_v, rows, gsems, ssems, acc,
                   wid < N_XTRA)

        plsc.subcore_barrier()
        pltpu.sync_copy(acc.at[pl.ds(s * ZROWS, ZROWS)],
                        out_hbm.at[c, pl.ds(s * ZROWS, ZROWS)])

    return spmm(parts, edges, zeros_hbm)


def _project(features, w1):
    def body(f_ref, w_ref, o_ref):
        o_ref[...] = jnp.dot(f_ref[...], w_ref[...],
                             preferred_element_type=jnp.float32)

    return pl.pallas_call(
        body,
        grid=(5,),
        in_specs=[
            pl.BlockSpec((2000, D_FEAT), lambda i: (i, 0)),
            pl.BlockSpec((D_FEAT, HID), lambda i: (0, 0)),
        ],
        out_specs=pl.BlockSpec((2000, HID), lambda i: (i, 0)),
        out_shape=jax.ShapeDtypeStruct((N_NODES, HID), jnp.float32),
    )(features, w1)


def _combine_matmul(parts, w3):
    def body(p_ref, w_ref, o_ref):
        z = p_ref[0] + p_ref[1]
        o_ref[...] = jnp.dot(z, w_ref[...],
                             preferred_element_type=jnp.float32)

    return pl.pallas_call(
        body,
        grid=(5,),
        in_specs=[
            pl.BlockSpec((NC, 2000, HID), lambda i: (0, i, 0)),
            pl.BlockSpec((HID, N_CLASSES), lambda i: (0, 0)),
        ],
        out_specs=pl.BlockSpec((2000, N_CLASSES), lambda i: (i, 0)),
        out_shape=jax.ShapeDtypeStruct((N_NODES, N_CLASSES), jnp.float32),
    )(parts, w3)


def kernel(features, edge_index, weight1, weight3):
    edges = edge_index.reshape(2, ROWS_E, CHUNK)
    zeros_hbm = jnp.zeros((ZROWS, HID), jnp.float32)

    x1 = _project(features, weight1)
    p1 = _spmm_sc1(x1, edges, zeros_hbm)
    p2 = _spmm_sc2(p1, edges, zeros_hbm)
    return _combine_matmul(p2, weight3)


# speedup vs baseline: 32.2768x; 1.1025x over previous
import functools

import jax
import jax.numpy as jnp
from jax import lax
from jax.experimental import pallas as pl
from jax.experimental.pallas import tpu as pltpu
from jax.experimental.pallas import tpu_sc as plsc

N_NODES = 10000
N_EDGES = 320000
D_FEAT = 128
HID = 16
N_CLASSES = 64

NC = 2
NS = 16
NW = NC * NS
CHUNK = 128
ROWS_E = N_EDGES // CHUNK
K_BASE = ROWS_E // NW
N_XTRA = ROWS_E - K_BASE * NW
N_ACC = 10240
ZROWS = N_ACC // NS
NBUF = 6
LAG = 3


def _spmm_body(x_src, src_v, dst_v, rows, gsems, ssems, acc, has_extra):
    def gather_wait(j, b):
        pltpu.make_async_copy(
            x_src.at[src_v.at[j]], rows.at[b], gsems.at[b]).wait()

    def scatter_wait(j, b):
        pltpu.make_async_copy(
            rows.at[b], acc.at[dst_v.at[j]], ssems.at[b]).wait()

    for b in range(LAG):
        pltpu.async_copy(x_src.at[src_v.at[b]], rows.at[b], gsems.at[b])

    def step(g, carry):
        for b in range(NBUF):
            j = g * NBUF + b
            gather_wait(j, b)
            pltpu.async_copy(rows.at[b], acc.at[dst_v.at[j]], ssems.at[b],
                             add=True)
            bn = (b + LAG) % NBUF

            @pl.when(j + LAG < K_BASE)
            def _():
                @pl.when(j >= LAG)
                def _():
                    scatter_wait(j - LAG, bn)

                pltpu.async_copy(
                    x_src.at[src_v.at[j + LAG]], rows.at[bn], gsems.at[bn])

        return carry

    lax.fori_loop(0, K_BASE // NBUF, step, 0)
    for j in range(K_BASE - 2 * LAG, K_BASE):
        scatter_wait(j, j % NBUF)

    @pl.when(has_extra)
    def _():
        pltpu.async_copy(x_src.at[src_v.at[K_BASE]], rows.at[0], gsems.at[0])
        gather_wait(K_BASE, 0)
        pltpu.async_copy(rows.at[0], acc.at[dst_v.at[K_BASE]], ssems.at[0],
                         add=True)
        scatter_wait(K_BASE, 0)


def _sc_scratch():
    return [
        pltpu.VMEM((K_BASE + 1, CHUNK), jnp.int32),
        pltpu.VMEM((K_BASE + 1, CHUNK), jnp.int32),
        pltpu.VMEM((NBUF, CHUNK, HID), jnp.float32),
        pltpu.VMEM_SHARED((N_ACC, HID), jnp.float32),
        pltpu.SemaphoreType.DMA((NBUF,)),
        pltpu.SemaphoreType.DMA((NBUF,)),
    ]


def _stage_indices(e_hbm, src_v, dst_v, wid):
    row0 = wid * K_BASE + jnp.minimum(wid, N_XTRA)
    pltpu.sync_copy(e_hbm.at[0, pl.ds(row0, K_BASE)],
                    src_v.at[pl.ds(0, K_BASE)])
    pltpu.sync_copy(e_hbm.at[1, pl.ds(row0, K_BASE)],
                    dst_v.at[pl.ds(0, K_BASE)])

    @pl.when(wid < N_XTRA)
    def _():
        pltpu.sync_copy(e_hbm.at[0, pl.ds(row0 + K_BASE, 1)],
                        src_v.at[pl.ds(K_BASE, 1)])
        pltpu.sync_copy(e_hbm.at[1, pl.ds(row0 + K_BASE, 1)],
                        dst_v.at[pl.ds(K_BASE, 1)])


def _spmm_sc1(x, edges, zeros_hbm):
    mesh = plsc.VectorSubcoreMesh(core_axis_name="c", subcore_axis_name="s")

    @functools.partial(
        pl.kernel,
        mesh=mesh,
        out_type=jax.ShapeDtypeStruct((NC, N_ACC, HID), jnp.float32),
        scratch_types=_sc_scratch() + [
            pltpu.VMEM_SHARED((N_NODES, HID), jnp.float32),
        ],
        compiler_params=pltpu.CompilerParams(use_tc_tiling_on_sc=False),
    )
    def spmm(x_hbm, e_hbm, z_hbm, out_hbm,
             src_v, dst_v, rows, acc, gsems, ssems, xtab):
        c = lax.axis_index("c")
        s = lax.axis_index("s")
        wid = c * NS + s
        xrows = N_NODES // NS

        pltpu.sync_copy(z_hbm, acc.at[pl.ds(s * ZROWS, ZROWS)])
        pltpu.sync_copy(x_hbm.at[pl.ds(s * xrows, xrows)],
                        xtab.at[pl.ds(s * xrows, xrows)])
        _stage_indices(e_hbm, src_v, dst_v, wid)
        plsc.subcore_barrier()

        _spmm_body(xtab, src_v, dst_v, rows, gsems, ssems, acc,
                   wid < N_XTRA)

        plsc.subcore_barrier()
        pltpu.sync_copy(acc.at[pl.ds(s * ZROWS, ZROWS)],
                        out_hbm.at[c, pl.ds(s * ZROWS, ZROWS)])

    return spmm(x, edges, zeros_hbm)


def _spmm_sc2(parts, edges, zeros_hbm):
    mesh = plsc.VectorSubcoreMesh(core_axis_name="c", subcore_axis_name="s")

    @functools.partial(
        pl.kernel,
        mesh=mesh,
        out_type=jax.ShapeDtypeStruct((NC, N_ACC, HID), jnp.float32),
        scratch_types=_sc_scratch() + [
            pltpu.VMEM_SHARED((N_ACC, HID), jnp.float32),
            pltpu.VMEM((ZROWS, HID), jnp.float32),
            pltpu.VMEM((ZROWS, HID), jnp.float32),
        ],
        compiler_params=pltpu.CompilerParams(use_tc_tiling_on_sc=False),
    )
    def spmm(p_hbm, e_hbm, z_hbm, out_hbm,
             src_v, dst_v, rows, acc, gsems, ssems, htab, p0_v, p1_v):
        c = lax.axis_index("c")
        s = lax.axis_index("s")
        wid = c * NS + s

        pltpu.sync_copy(z_hbm, acc.at[pl.ds(s * ZROWS, ZROWS)])
        _stage_indices(e_hbm, src_v, dst_v, wid)

        pltpu.sync_copy(p_hbm.at[0, pl.ds(s * ZROWS, ZROWS)], p0_v)
        pltpu.sync_copy(p_hbm.at[1, pl.ds(s * ZROWS, ZROWS)], p1_v)

        def relu_row(i, carry):
            p0_v[i, :] = jnp.maximum(p0_v[i, :] + p1_v[i, :], 0.0)
            return carry

        lax.fori_loop(0, ZROWS, relu_row, 0)
        pltpu.sync_copy(p0_v, htab.at[pl.ds(s * ZROWS, ZROWS)])
        plsc.subcore_barrier()

        _spmm_body(htab, src_v, dst_v, rows, gsems, ssems, acc,
                   wid < N_XTRA)

        plsc.subcore_barrier()
        pltpu.sync_copy(acc.at[pl.ds(s * ZROWS, ZROWS)],
                        out_hbm.at[c, pl.ds(s * ZROWS, ZROWS)])

    return spmm(parts, edges, zeros_hbm)


def _project(features, w1):
    def body(f_ref, w_ref, o_ref):
        o_ref[...] = jnp.dot(f_ref[...], w_ref[...],
                             preferred_element_type=jnp.float32)

    return pl.pallas_call(
        body,
        grid=(5,),
        in_specs=[
            pl.BlockSpec((2000, D_FEAT), lambda i: (i, 0)),
            pl.BlockSpec((D_FEAT, HID), lambda i: (0, 0)),
        ],
        out_specs=pl.BlockSpec((2000, HID), lambda i: (i, 0)),
        out_shape=jax.ShapeDtypeStruct((N_NODES, HID), jnp.float32),
    )(features, w1)


def _combine_matmul(parts, w3):
    def body(p_ref, w_ref, o_ref):
        z = p_ref[0] + p_ref[1]
        o_ref[...] = jnp.dot(z, w_ref[...],
                             preferred_element_type=jnp.float32)

    return pl.pallas_call(
        body,
        grid=(5,),
        in_specs=[
            pl.BlockSpec((NC, 2000, HID), lambda i: (0, i, 0)),
            pl.BlockSpec((HID, N_CLASSES), lambda i: (0, 0)),
        ],
        out_specs=pl.BlockSpec((2000, N_CLASSES), lambda i: (i, 0)),
        out_shape=jax.ShapeDtypeStruct((N_NODES, N_CLASSES), jnp.float32),
    )(parts, w3)


def kernel(features, edge_index, weight1, weight3):
    edges = edge_index.reshape(2, ROWS_E, CHUNK)
    zeros_hbm = jnp.zeros((ZROWS, HID), jnp.float32)

    x1 = _project(features, weight1)
    p1 = _spmm_sc1(x1, edges, zeros_hbm)
    p2 = _spmm_sc2(p1, edges, zeros_hbm)
    return _combine_matmul(p2, weight3)
